# Initial kernel scaffold; baseline (speedup 1.0000x reference)
import jax
import jax.numpy as jnp
from jax.experimental import pallas as pl


def kernel(hidden_states, gate_w, sg_w, su_w, sd_w, rg_w, ru_w, rd_w):
    raise NotImplementedError("write your pallas kernel here")


# speedup vs baseline: 1.2482x; 1.2482x over previous
import functools

import jax
import jax.numpy as jnp
from jax import lax
from jax.experimental import pallas as pl
from jax.experimental.pallas import tpu as pltpu
from jax.experimental.pallas import tpu_sc as plsc


def _sigmoid(x):
    return 1.0 / (1.0 + jnp.exp(-x))


def _k1_body(x_ref, gate_ref, sg_ref, su_ref, sd_ref, logits_ref, hsx_ref, shared_ref):
    x = x_ref[...]
    dn = (((1,), (1,)), ((), ()))
    logits = lax.dot_general(x, gate_ref[...], dn, preferred_element_type=jnp.float32)
    logits_ref[...] = logits
    score = _sigmoid(jnp.max(logits, axis=1, keepdims=True))
    hsx_ref[...] = x * score
    g = lax.dot_general(x, sg_ref[...], dn, preferred_element_type=jnp.float32)
    u = lax.dot_general(x, su_ref[...], dn, preferred_element_type=jnp.float32)
    h = g * _sigmoid(g) * u
    shared_ref[...] = lax.dot_general(h, sd_ref[...], dn, preferred_element_type=jnp.float32)


def _k3_body(expert_sel, valid, x_ref, sh_ref, rg_ref, ru_ref, rd_ref, out_ref):
    i = pl.program_id(0)
    dn = (((1,), (1,)), ((), ()))

    @pl.when(valid[i] > 0)
    def _():
        x = x_ref[...]
        g = lax.dot_general(x, rg_ref[0], dn, preferred_element_type=jnp.float32)
        u = lax.dot_general(x, ru_ref[0], dn, preferred_element_type=jnp.float32)
        h = g * _sigmoid(g) * u
        out_ref[...] = (
            lax.dot_general(h, rd_ref[0], dn, preferred_element_type=jnp.float32)
            + sh_ref[...]
        )


def kernel(hidden_states, gate_w, sg_w, su_w, sd_w, rg_w, ru_w, rd_w):
    B_, S_, H_ = hidden_states.shape
    E_ = gate_w.shape[0]
    Ish = sg_w.shape[0]
    Ir = rg_w.shape[1]
    T_ = B_ * S_

    TM1 = 256
    TM = 128
    NP = T_ // TM + E_
    NROWS = NP * TM

    hs2 = hidden_states.reshape(T_, H_)

    logits, hsx, shared = pl.pallas_call(
        _k1_body,
        grid=(T_ // TM1,),
        in_specs=[
            pl.BlockSpec((TM1, H_), lambda i: (i, 0)),
            pl.BlockSpec((E_, H_), lambda i: (0, 0)),
            pl.BlockSpec((Ish, H_), lambda i: (0, 0)),
            pl.BlockSpec((Ish, H_), lambda i: (0, 0)),
            pl.BlockSpec((H_, Ish), lambda i: (0, 0)),
        ],
        out_specs=[
            pl.BlockSpec((TM1, E_), lambda i: (i, 0)),
            pl.BlockSpec((TM1, H_), lambda i: (i, 0)),
            pl.BlockSpec((TM1, H_), lambda i: (i, 0)),
        ],
        out_shape=[
            jax.ShapeDtypeStruct((T_, E_), jnp.float32),
            jax.ShapeDtypeStruct((T_, H_), jnp.float32),
            jax.ShapeDtypeStruct((T_, H_), jnp.float32),
        ],
    )(hs2, gate_w, sg_w, su_w, sd_w)

    eid = jnp.argmax(logits, axis=1).astype(jnp.int32)
    perm = jnp.argsort(eid).astype(jnp.int32)
    es = eid[perm]
    sizes = jnp.zeros((E_,), jnp.int32).at[eid].add(1)
    offsets = jnp.cumsum(sizes) - sizes
    tiles_per = (sizes + TM - 1) // TM
    tile_start = (jnp.cumsum(tiles_per) - tiles_per).astype(jnp.int32)
    used_tiles = jnp.sum(tiles_per)

    k = jnp.arange(T_, dtype=jnp.int32)
    rank = k - offsets[es]
    pos = tile_start[es] * TM + rank
    idx_pad = jnp.zeros((NROWS,), jnp.int32).at[pos].set(perm)
    inv_pos = jnp.zeros((T_,), jnp.int32).at[perm].set(pos)

    tl = jnp.arange(NP, dtype=jnp.int32)
    e_raw = (jnp.searchsorted(tile_start, tl, side="right") - 1).astype(jnp.int32)
    valid = jnp.clip(sizes[e_raw] - (tl - tile_start[e_raw]) * TM, 0, TM).astype(jnp.int32)
    tl_c = jnp.minimum(tl, used_tiles - 1)
    expert_sel = (jnp.searchsorted(tile_start, tl_c, side="right") - 1).astype(jnp.int32)

    NC, NSC = 2, 16
    NW = NC * NSC
    mesh = plsc.VectorSubcoreMesh(
        core_axis_name="c", subcore_axis_name="s", num_cores=NC, num_subcores=NSC
    )

    rows_w = NROWS // NW
    CG = 16
    nch = rows_w // CG

    @functools.partial(
        pl.kernel,
        out_type=(
            jax.ShapeDtypeStruct((NROWS, H_), jnp.float32),
            jax.ShapeDtypeStruct((NROWS, H_), jnp.float32),
        ),
        mesh=mesh,
        scratch_types=[
            pltpu.VMEM((CG,), jnp.int32),
            pltpu.VMEM((CG, H_), jnp.float32),
            pltpu.VMEM((CG, H_), jnp.float32),
            pltpu.SemaphoreType.DMA,
            pltpu.SemaphoreType.DMA,
        ],
    )
    def _gather2(hsx_hbm, sh_hbm, idx_hbm, xo_hbm, so_hbm, idx_v, ba, bb, sa, sb):
        wid = lax.axis_index("s") * NC + lax.axis_index("c")
        base = wid * rows_w

        def body(ci, carry):
            start = base + ci * CG
            pltpu.sync_copy(idx_hbm.at[pl.ds(start, CG)], idx_v)
            c1 = pltpu.async_copy(hsx_hbm.at[idx_v], ba, sa)
            c2 = pltpu.async_copy(sh_hbm.at[idx_v], bb, sb)
            c1.wait()
            c2.wait()
            pltpu.sync_copy(ba, xo_hbm.at[pl.ds(start, CG)])
            pltpu.sync_copy(bb, so_hbm.at[pl.ds(start, CG)])
            return carry

        lax.fori_loop(0, nch, body, 0)

    x_pad, sh_pad = _gather2(hsx, shared, idx_pad)

    final_pad = pl.pallas_call(
        _k3_body,
        grid_spec=pltpu.PrefetchScalarGridSpec(
            num_scalar_prefetch=2,
            grid=(NP,),
            in_specs=[
                pl.BlockSpec((TM, H_), lambda i, e, v: (i, 0)),
                pl.BlockSpec((TM, H_), lambda i, e, v: (i, 0)),
                pl.BlockSpec((1, Ir, H_), lambda i, e, v: (e[i], 0, 0)),
                pl.BlockSpec((1, Ir, H_), lambda i, e, v: (e[i], 0, 0)),
                pl.BlockSpec((1, H_, Ir), lambda i, e, v: (e[i], 0, 0)),
            ],
            out_specs=pl.BlockSpec((TM, H_), lambda i, e, v: (i, 0)),
        ),
        out_shape=jax.ShapeDtypeStruct((NROWS, H_), jnp.float32),
    )(expert_sel, valid, x_pad, sh_pad, rg_w, ru_w, rd_w)

    rows4_w = T_ // NW
    C4 = 32
    nch4 = rows4_w // C4

    @functools.partial(
        pl.kernel,
        out_type=jax.ShapeDtypeStruct((T_, H_), jnp.float32),
        mesh=mesh,
        scratch_types=[
            pltpu.VMEM((C4,), jnp.int32),
            pltpu.VMEM((C4, H_), jnp.float32),
            pltpu.SemaphoreType.DMA,
        ],
    )
    def _gather_fin(fin_hbm, inv_hbm, out_hbm, idx_v, buf, sem):
        wid = lax.axis_index("s") * NC + lax.axis_index("c")
        base = wid * rows4_w

        def body(ci, carry):
            start = base + ci * C4
            pltpu.sync_copy(inv_hbm.at[pl.ds(start, C4)], idx_v)
            pltpu.async_copy(fin_hbm.at[idx_v], buf, sem).wait()
            pltpu.sync_copy(buf, out_hbm.at[pl.ds(start, C4)])
            return carry

        lax.fori_loop(0, nch4, body, 0)

    out2 = _gather_fin(final_pad, inv_pos)

    return (out2.reshape(B_, S_, H_), logits.reshape(B_, S_, E_))


# speedup vs baseline: 1.2875x; 1.0315x over previous
import functools

import jax
import jax.numpy as jnp
from jax import lax
from jax.experimental import pallas as pl
from jax.experimental.pallas import tpu as pltpu
from jax.experimental.pallas import tpu_sc as plsc


def _sigmoid(x):
    return 1.0 / (1.0 + jnp.exp(-x))


def _k1_body(x_ref, gate_ref, sg_ref, su_ref, sd_ref, logits_ref, hsx_ref, shared_ref):
    x = x_ref[...]
    dn = (((1,), (1,)), ((), ()))
    logits = lax.dot_general(x, gate_ref[...], dn, preferred_element_type=jnp.float32)
    logits_ref[...] = logits
    score = _sigmoid(jnp.max(logits, axis=1, keepdims=True))
    hsx_ref[...] = x * score
    g = lax.dot_general(x, sg_ref[...], dn, preferred_element_type=jnp.float32)
    u = lax.dot_general(x, su_ref[...], dn, preferred_element_type=jnp.float32)
    h = g * _sigmoid(g) * u
    shared_ref[...] = lax.dot_general(h, sd_ref[...], dn, preferred_element_type=jnp.float32)


def _k3_body(expert_sel, valid, x_ref, sh_ref, rg_ref, ru_ref, rd_ref, out_ref):
    i = pl.program_id(0)
    dn = (((1,), (1,)), ((), ()))

    @pl.when(valid[i] > 0)
    def _():
        x = x_ref[...]
        g = lax.dot_general(x, rg_ref[0], dn, preferred_element_type=jnp.float32)
        u = lax.dot_general(x, ru_ref[0], dn, preferred_element_type=jnp.float32)
        h = g * _sigmoid(g) * u
        out_ref[...] = (
            lax.dot_general(h, rd_ref[0], dn, preferred_element_type=jnp.float32)
            + sh_ref[...]
        )


def kernel(hidden_states, gate_w, sg_w, su_w, sd_w, rg_w, ru_w, rd_w):
    B_, S_, H_ = hidden_states.shape
    E_ = gate_w.shape[0]
    Ish = sg_w.shape[0]
    Ir = rg_w.shape[1]
    T_ = B_ * S_

    TM1 = 256
    TM = 128
    NP = T_ // TM + E_
    NROWS = NP * TM

    hs2 = hidden_states.reshape(T_, H_)

    logits, hsx, shared = pl.pallas_call(
        _k1_body,
        grid=(T_ // TM1,),
        in_specs=[
            pl.BlockSpec((TM1, H_), lambda i: (i, 0)),
            pl.BlockSpec((E_, H_), lambda i: (0, 0)),
            pl.BlockSpec((Ish, H_), lambda i: (0, 0)),
            pl.BlockSpec((Ish, H_), lambda i: (0, 0)),
            pl.BlockSpec((H_, Ish), lambda i: (0, 0)),
        ],
        out_specs=[
            pl.BlockSpec((TM1, E_), lambda i: (i, 0)),
            pl.BlockSpec((TM1, H_), lambda i: (i, 0)),
            pl.BlockSpec((TM1, H_), lambda i: (i, 0)),
        ],
        out_shape=[
            jax.ShapeDtypeStruct((T_, E_), jnp.float32),
            jax.ShapeDtypeStruct((T_, H_), jnp.float32),
            jax.ShapeDtypeStruct((T_, H_), jnp.float32),
        ],
    )(hs2, gate_w, sg_w, su_w, sd_w)

    eid = jnp.argmax(logits, axis=1).astype(jnp.int32)
    oh = (eid[:, None] == jnp.arange(E_, dtype=jnp.int32)[None, :]).astype(jnp.int32)
    csum = jnp.cumsum(oh, axis=0)
    sizes = csum[-1]
    rank = jnp.take_along_axis(csum, eid[:, None], axis=1)[:, 0] - 1
    tiles_per = (sizes + TM - 1) // TM
    tile_start = (jnp.cumsum(tiles_per) - tiles_per).astype(jnp.int32)
    used_tiles = jnp.sum(tiles_per)

    pos_tok = tile_start[eid] * TM + rank
    idx_pad = jnp.zeros((NROWS,), jnp.int32).at[pos_tok].set(
        jnp.arange(T_, dtype=jnp.int32))
    inv_pos = pos_tok

    tl = jnp.arange(NP, dtype=jnp.int32)
    e_raw = (jnp.searchsorted(tile_start, tl, side="right") - 1).astype(jnp.int32)
    valid = jnp.clip(sizes[e_raw] - (tl - tile_start[e_raw]) * TM, 0, TM).astype(jnp.int32)
    tl_c = jnp.minimum(tl, used_tiles - 1)
    expert_sel = (jnp.searchsorted(tile_start, tl_c, side="right") - 1).astype(jnp.int32)

    NC, NSC = 2, 16
    NW = NC * NSC
    mesh = plsc.VectorSubcoreMesh(
        core_axis_name="c", subcore_axis_name="s", num_cores=NC, num_subcores=NSC
    )

    rows_w = NROWS // NW
    CG = 8
    NB = 3
    n_outer = rows_w // (CG * NB)

    @functools.partial(
        pl.kernel,
        out_type=(
            jax.ShapeDtypeStruct((NROWS, H_), jnp.float32),
            jax.ShapeDtypeStruct((NROWS, H_), jnp.float32),
        ),
        mesh=mesh,
        scratch_types=(
            [pltpu.VMEM((rows_w,), jnp.int32)]
            + [pltpu.VMEM((CG, H_), jnp.float32)] * (2 * NB)
            + [pltpu.SemaphoreType.DMA] * (4 * NB)
        ),
    )
    def _gather2(hsx_hbm, sh_hbm, idx_hbm, xo_hbm, so_hbm, idx_v, *sc):
        bufs = (sc[0:NB], sc[NB:2 * NB])
        gs = (sc[2 * NB:3 * NB], sc[3 * NB:4 * NB])
        ws = (sc[4 * NB:5 * NB], sc[5 * NB:6 * NB])
        tabs = (hsx_hbm, sh_hbm)
        outs = (xo_hbm, so_hbm)
        wid = lax.axis_index("s") * NC + lax.axis_index("c")
        base = wid * rows_w
        pltpu.sync_copy(idx_hbm.at[pl.ds(base, rows_w)], idx_v)

        def body(j, carry):
            c0 = j * NB
            cps = []
            for b in range(NB):
                sl = pl.ds((c0 + b) * CG, CG)
                for t in range(2):
                    cps.append((t, b, pltpu.async_copy(
                        tabs[t].at[idx_v.at[sl]], bufs[t][b], gs[t][b])))
            wcps = []
            for t, b, cp in cps:
                cp.wait()
                osl = pl.ds(base + (c0 + b) * CG, CG)
                wcps.append(pltpu.async_copy(bufs[t][b], outs[t].at[osl], ws[t][b]))
            for wcp in wcps:
                wcp.wait()
            return carry

        lax.fori_loop(0, n_outer, body, 0)

    x_pad, sh_pad = _gather2(hsx, shared, idx_pad)

    final_pad = pl.pallas_call(
        _k3_body,
        grid_spec=pltpu.PrefetchScalarGridSpec(
            num_scalar_prefetch=2,
            grid=(NP,),
            in_specs=[
                pl.BlockSpec((TM, H_), lambda i, e, v: (i, 0)),
                pl.BlockSpec((TM, H_), lambda i, e, v: (i, 0)),
                pl.BlockSpec((1, Ir, H_), lambda i, e, v: (e[i], 0, 0)),
                pl.BlockSpec((1, Ir, H_), lambda i, e, v: (e[i], 0, 0)),
                pl.BlockSpec((1, H_, Ir), lambda i, e, v: (e[i], 0, 0)),
            ],
            out_specs=pl.BlockSpec((TM, H_), lambda i, e, v: (i, 0)),
        ),
        out_shape=jax.ShapeDtypeStruct((NROWS, H_), jnp.float32),
    )(expert_sel, valid, x_pad, sh_pad, rg_w, ru_w, rd_w)

    rows4_w = T_ // NW
    C4 = 16
    NB4 = 2
    n_outer4 = rows4_w // (C4 * NB4)

    @functools.partial(
        pl.kernel,
        out_type=jax.ShapeDtypeStruct((T_, H_), jnp.float32),
        mesh=mesh,
        scratch_types=(
            [pltpu.VMEM((rows4_w,), jnp.int32)]
            + [pltpu.VMEM((C4, H_), jnp.float32)] * NB4
            + [pltpu.SemaphoreType.DMA] * (2 * NB4)
        ),
    )
    def _gather_fin(fin_hbm, inv_hbm, out_hbm, idx_v, *sc):
        bufs = sc[0:NB4]
        gs = sc[NB4:2 * NB4]
        ws = sc[2 * NB4:3 * NB4]
        wid = lax.axis_index("s") * NC + lax.axis_index("c")
        base = wid * rows4_w
        pltpu.sync_copy(inv_hbm.at[pl.ds(base, rows4_w)], idx_v)

        def body(j, carry):
            c0 = j * NB4
            cps = []
            for b in range(NB4):
                sl = pl.ds((c0 + b) * C4, C4)
                cps.append((b, pltpu.async_copy(
                    fin_hbm.at[idx_v.at[sl]], bufs[b], gs[b])))
            wcps = []
            for b, cp in cps:
                cp.wait()
                osl = pl.ds(base + (c0 + b) * C4, C4)
                wcps.append(pltpu.async_copy(bufs[b], out_hbm.at[osl], ws[b]))
            for wcp in wcps:
                wcp.wait()
            return carry

        lax.fori_loop(0, n_outer4, body, 0)

    out2 = _gather_fin(final_pad, inv_pos)

    return (out2.reshape(B_, S_, H_), logits.reshape(B_, S_, E_))


# speedup vs baseline: 1.7461x; 1.3562x over previous
import functools

import jax
import jax.numpy as jnp
from jax import lax
from jax.experimental import pallas as pl
from jax.experimental.pallas import tpu as pltpu
from jax.experimental.pallas import tpu_sc as plsc


def _sigmoid(x):
    return 1.0 / (1.0 + jnp.exp(-x))


def _k1_body(x_ref, gate_ref, sg_ref, su_ref, sd_ref, logits_ref, hsx_ref, shared_ref):
    x = x_ref[...]
    dn = (((1,), (1,)), ((), ()))
    logits = lax.dot_general(x, gate_ref[...], dn, preferred_element_type=jnp.float32)
    logits_ref[...] = logits
    score = _sigmoid(jnp.max(logits, axis=1, keepdims=True))
    hsx_ref[...] = x * score
    g = lax.dot_general(x, sg_ref[...], dn, preferred_element_type=jnp.float32)
    u = lax.dot_general(x, su_ref[...], dn, preferred_element_type=jnp.float32)
    h = g * _sigmoid(g) * u
    shared_ref[...] = lax.dot_general(h, sd_ref[...], dn, preferred_element_type=jnp.float32)


def _make_k3_body(TM):
    def _k3_body(step_t, step_e, step_lo, step_hi, x_ref, sh_ref, rg_ref, ru_ref,
                 rd_ref, out_ref):
        s = pl.program_id(0)
        lo = step_lo[s]
        hi = step_hi[s]
        dn = (((1,), (1,)), ((), ()))

        @pl.when(lo < hi)
        def _():
            x = x_ref[...]
            g = lax.dot_general(x, rg_ref[0], dn, preferred_element_type=jnp.float32)
            u = lax.dot_general(x, ru_ref[0], dn, preferred_element_type=jnp.float32)
            h = g * _sigmoid(g) * u
            y = (
                lax.dot_general(h, rd_ref[0], dn, preferred_element_type=jnp.float32)
                + sh_ref[...]
            )
            rows = lax.broadcasted_iota(jnp.int32, (x.shape[0], 1), 0)
            mask = (rows >= lo) & (rows < hi)
            out_ref[...] = jnp.where(mask, y, out_ref[...])

    return _k3_body


def kernel(hidden_states, gate_w, sg_w, su_w, sd_w, rg_w, ru_w, rd_w):
    B_, S_, H_ = hidden_states.shape
    E_ = gate_w.shape[0]
    Ish = sg_w.shape[0]
    Ir = rg_w.shape[1]
    T_ = B_ * S_

    TM1 = 256
    TM = 128
    NT = T_ // TM
    NSTEP = NT + E_ - 1

    hs2 = hidden_states.reshape(T_, H_)

    logits, hsx, shared = pl.pallas_call(
        _k1_body,
        grid=(T_ // TM1,),
        in_specs=[
            pl.BlockSpec((TM1, H_), lambda i: (i, 0)),
            pl.BlockSpec((E_, H_), lambda i: (0, 0)),
            pl.BlockSpec((Ish, H_), lambda i: (0, 0)),
            pl.BlockSpec((Ish, H_), lambda i: (0, 0)),
            pl.BlockSpec((H_, Ish), lambda i: (0, 0)),
        ],
        out_specs=[
            pl.BlockSpec((TM1, E_), lambda i: (i, 0)),
            pl.BlockSpec((TM1, H_), lambda i: (i, 0)),
            pl.BlockSpec((TM1, H_), lambda i: (i, 0)),
        ],
        out_shape=[
            jax.ShapeDtypeStruct((T_, E_), jnp.float32),
            jax.ShapeDtypeStruct((T_, H_), jnp.float32),
            jax.ShapeDtypeStruct((T_, H_), jnp.float32),
        ],
    )(hs2, gate_w, sg_w, su_w, sd_w)

    eid = jnp.argmax(logits, axis=1).astype(jnp.int32)
    oh = (eid[:, None] == jnp.arange(E_, dtype=jnp.int32)[None, :]).astype(jnp.int32)
    csum = jnp.cumsum(oh, axis=0)
    sizes = csum[-1]
    rank = jnp.take_along_axis(csum, eid[:, None], axis=1)[:, 0] - 1
    offsets = jnp.cumsum(sizes) - sizes
    pos_tok = offsets[eid] + rank
    idx_sorted = jnp.zeros((T_,), jnp.int32).at[pos_tok].set(
        jnp.arange(T_, dtype=jnp.int32))
    inv_pos = pos_tok

    ends = offsets + sizes
    t_all = jnp.arange(NT, dtype=jnp.int32)[:, None]
    e_all = jnp.arange(E_, dtype=jnp.int32)[None, :]
    lo2 = jnp.maximum(offsets[None, :], t_all * TM)
    hi2 = jnp.minimum(ends[None, :], (t_all + 1) * TM)
    fvalid = (lo2 < hi2).reshape(-1)
    srank = jnp.cumsum(fvalid.astype(jnp.int32)) - 1
    sidx = jnp.where(fvalid, srank, NSTEP)
    tt = jnp.broadcast_to(t_all, (NT, E_)).reshape(-1)
    ee = jnp.broadcast_to(e_all, (NT, E_)).reshape(-1)
    e_last = jnp.max(jnp.where(sizes > 0, jnp.arange(E_, dtype=jnp.int32), -1))
    step_t = jnp.full((NSTEP,), NT - 1, jnp.int32).at[sidx].set(tt, mode="drop")
    step_e = (jnp.zeros((NSTEP,), jnp.int32) + e_last).at[sidx].set(ee, mode="drop")
    step_lo = jnp.zeros((NSTEP,), jnp.int32).at[sidx].set(
        (lo2 - t_all * TM).reshape(-1), mode="drop")
    step_hi = jnp.zeros((NSTEP,), jnp.int32).at[sidx].set(
        (hi2 - t_all * TM).reshape(-1), mode="drop")

    NC, NSC = 2, 16
    NW = NC * NSC
    mesh = plsc.VectorSubcoreMesh(
        core_axis_name="c", subcore_axis_name="s", num_cores=NC, num_subcores=NSC
    )

    def sc_gather(tables, idx):
        ntab = len(tables)
        nrows = idx.shape[0]
        rows_w = nrows // NW
        CG = 32
        nch = rows_w // CG

        @functools.partial(
            pl.kernel,
            out_type=tuple(
                jax.ShapeDtypeStruct((nrows, H_), jnp.float32) for _ in range(ntab)
            ),
            mesh=mesh,
            scratch_types=[
                pltpu.VMEM((rows_w,), jnp.int32),
                pltpu.VMEM((CG, H_), jnp.float32),
                pltpu.SemaphoreType.DMA,
            ],
        )
        def _g(*refs):
            tabs = refs[:ntab]
            idx_hbm = refs[ntab]
            outs = refs[ntab + 1:2 * ntab + 1]
            idx_v, buf, sem = refs[2 * ntab + 1:]
            wid = lax.axis_index("s") * NC + lax.axis_index("c")
            base = wid * rows_w
            pltpu.sync_copy(idx_hbm.at[pl.ds(base, rows_w)], idx_v)

            def body(ci, carry):
                sl = pl.ds(ci * CG, CG)
                for t in range(ntab):
                    pltpu.async_copy(tabs[t].at[idx_v.at[sl]], buf, sem).wait()
                    pltpu.sync_copy(buf, outs[t].at[pl.ds(base + ci * CG, CG)])
                return carry

            lax.fori_loop(0, nch, body, 0)

        return _g(*tables, idx)

    x_srt, sh_srt = sc_gather((hsx, shared), idx_sorted)

    final_srt = pl.pallas_call(
        _make_k3_body(TM),
        grid_spec=pltpu.PrefetchScalarGridSpec(
            num_scalar_prefetch=4,
            grid=(NSTEP,),
            in_specs=[
                pl.BlockSpec((TM, H_), lambda i, st, se, sl, sh: (st[i], 0)),
                pl.BlockSpec((TM, H_), lambda i, st, se, sl, sh: (st[i], 0)),
                pl.BlockSpec((1, Ir, H_), lambda i, st, se, sl, sh: (se[i], 0, 0)),
                pl.BlockSpec((1, Ir, H_), lambda i, st, se, sl, sh: (se[i], 0, 0)),
                pl.BlockSpec((1, H_, Ir), lambda i, st, se, sl, sh: (se[i], 0, 0)),
            ],
            out_specs=pl.BlockSpec((TM, H_), lambda i, st, se, sl, sh: (st[i], 0)),
        ),
        out_shape=jax.ShapeDtypeStruct((T_, H_), jnp.float32),
    )(step_t, step_e, step_lo, step_hi, x_srt, sh_srt, rg_w, ru_w, rd_w)

    (out2,) = sc_gather((final_srt,), inv_pos)

    return (out2.reshape(B_, S_, H_), logits.reshape(B_, S_, E_))
